# baseline (speedup 1.0000x reference)
import jax
import jax.numpy as jnp
from jax import lax
from jax.experimental import pallas as pl
from jax.experimental.pallas import tpu as pltpu
from jax.experimental.pallas import tpu_sc as plsc

N_KEYS = 100000
BATCH = 16384
NS = 16
NC = 2
KPT = 6272
NPAD = NS * KPT
QPT = BATCH // (NC * NS)
LOGM = 19
M = 1 << LOGM
HASH_MUL = -1640531527
FILLW = 16384
CW = 512
CTHRESH = CW - 16
MAX_ROUNDS = 24


def _slot_of(hbits):
    return lax.shift_right_logical(hbits * HASH_MUL, 32 - LOGM)


def _hash_bits(h):
    return lax.convert_element_type(h * jnp.float32(1048576.0), jnp.int32)


def _scalar_total(vec):
    tot = jnp.int32(0)
    for l in range(16):
        tot = tot + vec[l]
    return tot


def _lookup_body(hk_hbm, hq_hbm, d_hbm, out_hbm,
                 neg1_v, hk_v, j_v, slot_v, won_v, wslot_v, rb_v,
                 cj_v, cslot_v, cwslot_v, crb_v,
                 posj_v, possl_v, cposj_v, cpossl_v, pfx32_v,
                 cnt_v, cntall_v,
                 qh_v, qslot_v, qj_v, qgi_v, qhg_v, qbest_v, qout_v,
                 table_sh, cnt_sh, cstage_sh):
    s = lax.axis_index("s")
    c = lax.axis_index("c")
    wid = s * NC + c
    soff_j = s * 2 * (CW + 16)
    soff_sl = soff_j + (CW + 16)

    def fill_body(i, carry):
        for u in range(8):
            neg1_v[pl.ds(i * 128 + u * 16, 16)] = jnp.full((16,), -1, jnp.int32)
        return carry
    lax.fori_loop(0, FILLW // 128, fill_body, 0)
    stripe = M // NS
    for r in range(stripe // FILLW):
        pltpu.sync_copy(neg1_v, table_sh.at[pl.ds(s * stripe + r * FILLW, FILLW)])

    pltpu.sync_copy(hk_hbm.at[pl.ds(s * KPT, KPT)], hk_v)

    def kinit(i, carry):
        for u in range(8):
            k = i * 8 + u
            ds16 = pl.ds(k * 16, 16)
            lane = lax.iota(jnp.int32, 16)
            j = s * KPT + k * 16 + lane
            sl = _slot_of(_hash_bits(hk_v[ds16]))
            valid = j < N_KEYS
            j_v[ds16] = j
            slot_v[ds16] = jnp.where(valid, sl, jnp.int32(M))
        return carry
    lax.fori_loop(0, KPT // 128, kinit, 0)
    cwslot_v[pl.ds(CW, 16)] = jnp.full((16,), M, jnp.int32)
    cslot_v[pl.ds(CW, 16)] = jnp.full((16,), M, jnp.int32)
    pfx32_v[pl.ds(0, 16)] = jnp.zeros((16,), jnp.int32)

    def _publish_and_total():
        pltpu.sync_copy(cnt_v, cnt_sh.at[pl.ds(s * 16, 16)])
        plsc.subcore_barrier()
        pltpu.sync_copy(cnt_sh, cntall_v)
        tot = jnp.zeros((16,), jnp.int32)
        for r in range(NS):
            tot = tot + cntall_v[pl.ds(r * 16, 16)]
        return _scalar_total(tot)

    def _prefix16(pending):
        v = jnp.where(pending, jnp.int32(1), jnp.int32(0))
        pend_i = v
        for sh in (1, 2, 4, 8):
            pfx32_v[pl.ds(16, 16)] = v
            v = v + pfx32_v[pl.ds(16 - sh, 16)]
        return v - pend_i, v[15]

    def _chunk_positions(k, pending, ptr):
        excl, cnt = _prefix16(pending)
        lane = lax.iota(jnp.int32, 16)
        loc = ptr + excl
        ok = pending & (loc <= CW - 1)
        pos = jnp.where(ok, loc, jnp.int32(CW) + lane)
        ds16 = pl.ds(k * 16, 16)
        posj_v[ds16] = soff_j + pos
        possl_v[ds16] = soff_sl + pos
        return ptr + cnt

    def _tail_park(ptr):
        def tp(i, carry):
            for u in range(8):
                k = i * 8 + u
                ds16 = pl.ds(k * 16, 16)
                pos = jnp.int32(k * 16) + lax.iota(jnp.int32, 16)
                cslot_v[ds16] = jnp.where(pos >= ptr, jnp.int32(M),
                                          cslot_v[ds16])
            return carry
        lax.fori_loop(0, CW // 128, tp, 0)
        cslot_v[pl.ds(CW, 16)] = jnp.full((16,), M, jnp.int32)

    def _gather_worklist_full():
        pltpu.sync_copy(j_v, cstage_sh.at[posj_v])
        pltpu.sync_copy(rb_v, cstage_sh.at[possl_v])
        pltpu.sync_copy(cstage_sh.at[pl.ds(soff_j, CW + 16)], cj_v)
        pltpu.sync_copy(cstage_sh.at[pl.ds(soff_sl, CW + 16)], cslot_v)

    plsc.subcore_barrier()
    pltpu.sync_copy(j_v, table_sh.at[slot_v])
    plsc.subcore_barrier()
    pltpu.sync_copy(table_sh.at[slot_v], rb_v)

    def r1_step(i, carry):
        ptr, pend = carry
        for u in range(8):
            k = i * 8 + u
            ds16 = pl.ds(k * 16, 16)
            sl = slot_v[ds16]
            jv = j_v[ds16]
            wn = (sl == M) | (rb_v[ds16] == jv)
            nsl = sl + 1
            nsl = jnp.where(nsl >= M, nsl - M, nsl)
            slot_v[ds16] = jnp.where(wn, sl, nsl)
            wni = jnp.where(wn, jnp.int32(1), jnp.int32(0))
            won_v[ds16] = wni
            pending = wni == 0
            rb_v[ds16] = nsl
            ptr = _chunk_positions(k, pending, ptr)
            pend = pend + jnp.where(pending, jnp.int32(1), jnp.int32(0))
        return ptr, pend
    wl1, pend1 = lax.fori_loop(0, KPT // 128, r1_step,
                               (jnp.int32(0), jnp.zeros((16,), jnp.int32)))
    _gather_worklist_full()
    _tail_park(wl1)
    cnt_v[...] = pend1
    total = _publish_and_total()

    def compact_round(wl_in):
        pltpu.sync_copy(table_sh.at[cslot_v], crb_v)

        def ca(i, carry):
            for u in range(8):
                k = i * 8 + u
                ds16 = pl.ds(k * 16, 16)
                sl = cslot_v[ds16]
                writer = (crb_v[ds16] == -1) & (sl != M)
                cwslot_v[ds16] = jnp.where(writer, sl, jnp.int32(M))
            return carry
        lax.fori_loop(0, CW // 128, ca, 0)
        plsc.subcore_barrier()
        pltpu.sync_copy(cj_v, table_sh.at[cwslot_v])
        plsc.subcore_barrier()
        pltpu.sync_copy(table_sh.at[cwslot_v], crb_v)

        def cb(i, carry):
            ptr, pend = carry
            for u in range(8):
                k = i * 8 + u
                ds16 = pl.ds(k * 16, 16)
                sl = cslot_v[ds16]
                jv = cj_v[ds16]
                ws = cwslot_v[ds16]
                wni = jnp.where((ws != M) & (crb_v[ds16] == jv),
                                jnp.int32(1), jnp.int32(0))
                pending = (wni == 0) & (sl != M)
                nsl = sl + 1
                nsl = jnp.where(nsl >= M, nsl - M, nsl)
                crb_v[ds16] = nsl
                excl, cnt = _prefix16(pending)
                lane = lax.iota(jnp.int32, 16)
                loc = ptr + excl
                ok = pending & (loc <= CW - 1)
                pos = jnp.where(ok, loc, jnp.int32(CW) + lane)
                cposj_v[ds16] = soff_j + pos
                cpossl_v[ds16] = soff_sl + pos
                ptr = ptr + cnt
                pend = pend + jnp.where(pending, jnp.int32(1), jnp.int32(0))
            return ptr, pend
        ptr, pend = lax.fori_loop(0, CW // 128, cb,
                                  (jnp.int32(0), jnp.zeros((16,), jnp.int32)))
        pltpu.sync_copy(cj_v.at[pl.ds(0, CW)], cstage_sh.at[cposj_v])
        pltpu.sync_copy(crb_v.at[pl.ds(0, CW)], cstage_sh.at[cpossl_v])
        pltpu.sync_copy(cstage_sh.at[pl.ds(soff_j, CW + 16)], cj_v)
        pltpu.sync_copy(cstage_sh.at[pl.ds(soff_sl, CW + 16)], cslot_v)
        _tail_park(ptr)
        cnt_v[...] = pend
        return ptr

    def full_round(wl_in):
        pltpu.sync_copy(table_sh.at[slot_v], rb_v)

        def fa(i, carry):
            for u in range(8):
                k = i * 8 + u
                ds16 = pl.ds(k * 16, 16)
                sl = slot_v[ds16]
                writer = (won_v[ds16] == 0) & (rb_v[ds16] == -1)
                wslot_v[ds16] = jnp.where(writer, sl, jnp.int32(M))
            return carry
        lax.fori_loop(0, KPT // 128, fa, 0)

        plsc.subcore_barrier()
        pltpu.sync_copy(j_v, table_sh.at[wslot_v])
        plsc.subcore_barrier()
        pltpu.sync_copy(table_sh.at[wslot_v], rb_v)

        def fb(i, carry):
            ptr, pend = carry
            for u in range(8):
                k = i * 8 + u
                ds16 = pl.ds(k * 16, 16)
                sl = slot_v[ds16]
                jv = j_v[ds16]
                wn = (won_v[ds16] == 1) | \
                     ((wslot_v[ds16] != M) & (rb_v[ds16] == jv))
                nsl = sl + 1
                nsl = jnp.where(nsl >= M, nsl - M, nsl)
                slot_v[ds16] = jnp.where(wn, sl, nsl)
                wni = jnp.where(wn, jnp.int32(1), jnp.int32(0))
                won_v[ds16] = wni
                pending = wni == 0
                rb_v[ds16] = nsl
                ptr = _chunk_positions(k, pending, ptr)
                pend = pend + jnp.where(pending, jnp.int32(1), jnp.int32(0))
            return ptr, pend
        ptr, pend = lax.fori_loop(0, KPT // 128, fb,
                                  (jnp.int32(0), jnp.zeros((16,), jnp.int32)))
        _gather_worklist_full()
        _tail_park(ptr)
        cnt_v[...] = pend
        return ptr

    def round_step(r, carry):
        wl_prev, tot_prev = carry

        def do_round(args):
            wl_p, _t = args
            wl_new = lax.cond(wl_p <= CTHRESH, compact_round,
                              full_round, wl_p)
            return wl_new, _publish_and_total()
        return lax.cond(tot_prev > 0, do_round, lambda x: x,
                        (wl_prev, tot_prev))

    lax.fori_loop(0, MAX_ROUNDS, round_step, (wl1, total))

    pltpu.sync_copy(hq_hbm.at[pl.ds(wid * QPT, QPT)], qh_v)

    def qinit(i, carry):
        for u in range(8):
            k = i * 8 + u
            ds16 = pl.ds(k * 16, 16)
            qslot_v[ds16] = _slot_of(_hash_bits(qh_v[ds16]))
            qbest_v[ds16] = jnp.full((16,), 0x7FFFFFFF, jnp.int32)
        return carry
    lax.fori_loop(0, QPT // 128, qinit, 0)

    def probe_body(_n):
        pltpu.sync_copy(table_sh.at[qslot_v], qj_v)

        def pg(i, carry):
            for u in range(8):
                k = i * 8 + u
                ds16 = pl.ds(k * 16, 16)
                qgi_v[ds16] = jnp.maximum(qj_v[ds16], jnp.int32(0))
            return carry
        lax.fori_loop(0, QPT // 128, pg, 0)
        pltpu.sync_copy(hk_hbm.at[qgi_v], qhg_v)

        def pu(i, nact):
            for u in range(8):
                k = i * 8 + u
                ds16 = pl.ds(k * 16, 16)
                jv = qj_v[ds16]
                act = qbest_v[ds16] >= 0
                sl = qslot_v[ds16]
                bst = qbest_v[ds16] & 0x7FFFFFFF
                match = act & (jv != -1) & (qhg_v[ds16] == qh_v[ds16])
                bst = jnp.where(match, jnp.minimum(bst, jv), bst)
                act_new = act & (jv != -1)
                qbest_v[ds16] = jnp.where(act_new, bst,
                                          bst | jnp.int32(-2147483648))
                nsl = sl + 1
                nsl = jnp.where(nsl >= M, nsl - M, nsl)
                qslot_v[ds16] = jnp.where(act_new, nsl, sl)
                nact = nact + jnp.where(act_new, jnp.int32(1), jnp.int32(0))
            return nact
        nact = lax.fori_loop(0, QPT // 128, pu, jnp.zeros((16,), jnp.int32))
        return _scalar_total(nact)

    def probe_step(r, n):
        return lax.cond(n > 0, probe_body, lambda x: x, n)
    lax.fori_loop(0, MAX_ROUNDS, probe_step, jnp.int32(1))

    def fin(i, carry):
        for u in range(8):
            k = i * 8 + u
            ds16 = pl.ds(k * 16, 16)
            qbest_v[ds16] = jnp.minimum(qbest_v[ds16] & 0x7FFFFFFF,
                                        jnp.int32(N_KEYS - 1))
        return carry
    lax.fori_loop(0, QPT // 128, fin, 0)

    pltpu.sync_copy(d_hbm.at[qbest_v], qout_v)
    pltpu.sync_copy(qout_v, out_hbm.at[wid])


@jax.jit
def _sc_lookup(hk_pad, h_in, d_flat):
    mesh = plsc.VectorSubcoreMesh(core_axis_name="c", subcore_axis_name="s")
    f = pl.kernel(
        _lookup_body,
        out_type=jax.ShapeDtypeStruct((NC * NS, QPT), jnp.float32),
        mesh=mesh,
        scratch_types=[
            pltpu.VMEM((FILLW,), jnp.int32),
            pltpu.VMEM((KPT,), jnp.float32),
            pltpu.VMEM((KPT,), jnp.int32),
            pltpu.VMEM((KPT,), jnp.int32),
            pltpu.VMEM((KPT,), jnp.int32),
            pltpu.VMEM((KPT,), jnp.int32),
            pltpu.VMEM((KPT,), jnp.int32),
            pltpu.VMEM((CW + 16,), jnp.int32),
            pltpu.VMEM((CW + 16,), jnp.int32),
            pltpu.VMEM((CW + 16,), jnp.int32),
            pltpu.VMEM((CW + 16,), jnp.int32),
            pltpu.VMEM((KPT,), jnp.int32),
            pltpu.VMEM((KPT,), jnp.int32),
            pltpu.VMEM((CW,), jnp.int32),
            pltpu.VMEM((CW,), jnp.int32),
            pltpu.VMEM((32,), jnp.int32),
            pltpu.VMEM((16,), jnp.int32),
            pltpu.VMEM((NS * 16,), jnp.int32),
            pltpu.VMEM((QPT,), jnp.float32),
            pltpu.VMEM((QPT,), jnp.int32),
            pltpu.VMEM((QPT,), jnp.int32),
            pltpu.VMEM((QPT,), jnp.int32),
            pltpu.VMEM((QPT,), jnp.float32),
            pltpu.VMEM((QPT,), jnp.int32),
            pltpu.VMEM((QPT,), jnp.float32),
            pltpu.VMEM_SHARED((M + 128,), jnp.int32),
            pltpu.VMEM_SHARED((NS * 16,), jnp.int32),
            pltpu.VMEM_SHARED((NS * 2 * (CW + 16),), jnp.int32),
        ],
    )
    return f(hk_pad, h_in, d_flat)


def kernel(inputs, keys_mat, distance_estimates, hash_vec):
    h_keys = jnp.sum(jnp.round(keys_mat, 5) * hash_vec, axis=-1)
    h_in = jnp.sum(jnp.round(inputs, 5) * hash_vec, axis=-1)
    hk_pad = jnp.pad(h_keys, (0, NPAD - N_KEYS))
    d_flat = distance_estimates[:, 0]
    out = _sc_lookup(hk_pad, h_in, d_flat)
    return out.reshape(BATCH, 1)


# speedup vs baseline: 1.7311x; 1.7311x over previous
import jax
import jax.numpy as jnp
from jax import lax
from jax.experimental import pallas as pl
from jax.experimental.pallas import tpu as pltpu
from jax.experimental.pallas import tpu_sc as plsc

N_KEYS = 100000
BATCH = 16384
NS = 16
NC = 2
KPT = 6272
NPAD = NS * KPT
QPT = BATCH // (NC * NS)
LOGM = 20
M = 1 << LOGM
HASH_MUL = -1640531527
NEG0 = -2147483648
FILLW = 16384
MAX_ROUNDS = 24


def _slot_of(hbits):
    return lax.shift_right_logical(hbits * HASH_MUL, 32 - LOGM)


def _hash_bits(h):
    return lax.convert_element_type(h * jnp.float32(1048576.0), jnp.int32)


def _scalar_total(vec):
    tot = jnp.int32(0)
    for l in range(16):
        tot = tot + vec[l]
    return tot


def _lookup_body(hk_hbm, hq_hbm, d_hbm, out_hbm,
                 neg1_v, hk_v, j_v, slot_v, won_v, wslot_v, rb_v,
                 cnt_v, cntall_v,
                 qh_v, qslot_v, qj_v, qgi_v, qhg_v, qbest_v, qout_v,
                 table_sh, cnt_sh):
    s = lax.axis_index("s")
    c = lax.axis_index("c")
    wid = s * NC + c

    def fill_body(i, carry):
        for u in range(8):
            neg1_v[pl.ds(i * 128 + u * 16, 16)] = jnp.full((16,), -1, jnp.int32)
        return carry
    lax.fori_loop(0, FILLW // 128, fill_body, 0)
    stripe = M // NS
    for r in range(stripe // FILLW):
        pltpu.sync_copy(neg1_v, table_sh.at[pl.ds(s * stripe + r * FILLW, FILLW)])

    pltpu.sync_copy(hk_hbm.at[pl.ds(s * KPT, KPT)], hk_v)

    def kinit(i, carry):
        for u in range(8):
            k = i * 8 + u
            ds16 = pl.ds(k * 16, 16)
            lane = lax.iota(jnp.int32, 16)
            j = s * KPT + k * 16 + lane
            sl = _slot_of(_hash_bits(hk_v[ds16]))
            valid = j < N_KEYS
            j_v[ds16] = j
            slot_v[ds16] = jnp.where(valid, sl, jnp.int32(M))
            won_v[ds16] = jnp.where(valid, jnp.int32(0), jnp.int32(1))
        return carry
    lax.fori_loop(0, KPT // 128, kinit, 0)

    plsc.subcore_barrier()

    def round_body(_tot):
        pltpu.sync_copy(table_sh.at[slot_v], rb_v)

        def a_step(i, carry):
            for u in range(8):
                ds16 = pl.ds(i * 128 + u * 16, 16)
                writer = (won_v[ds16] == 1) | (rb_v[ds16] == -1)
                wslot_v[ds16] = jnp.where(writer, slot_v[ds16], jnp.int32(M))
            return carry
        lax.fori_loop(0, KPT // 128, a_step, 0)

        plsc.subcore_barrier()
        pltpu.sync_copy(j_v, table_sh.at[wslot_v])
        plsc.subcore_barrier()
        pltpu.sync_copy(table_sh.at[wslot_v], rb_v)

        def b_step(i, acc):
            for u in range(8):
                ds16 = pl.ds(i * 128 + u * 16, 16)
                sl = slot_v[ds16]
                wn = (won_v[ds16] == 1) | ((wslot_v[ds16] != M) & (rb_v[ds16] == j_v[ds16]))
                nsl = sl + 1
                nsl = jnp.where(nsl >= M, nsl - M, nsl)
                slot_v[ds16] = jnp.where(wn, sl, nsl)
                won_v[ds16] = jnp.where(wn, jnp.int32(1), jnp.int32(0))
                acc = acc + jnp.where(wn, jnp.int32(0), jnp.int32(1))
            return acc
        pend = lax.fori_loop(0, KPT // 128, b_step, jnp.zeros((16,), jnp.int32))

        cnt_v[...] = pend
        pltpu.sync_copy(cnt_v, cnt_sh.at[pl.ds(s * 16, 16)])
        plsc.subcore_barrier()
        pltpu.sync_copy(cnt_sh, cntall_v)
        tot = jnp.zeros((16,), jnp.int32)
        for r in range(NS):
            tot = tot + cntall_v[pl.ds(r * 16, 16)]
        return _scalar_total(tot)

    def round_step(r, t):
        return lax.cond(t > 0, round_body, lambda x: x, t)
    lax.fori_loop(0, MAX_ROUNDS, round_step, jnp.int32(N_KEYS))

    pltpu.sync_copy(hq_hbm.at[pl.ds(wid * QPT, QPT)], qh_v)

    def qinit(k, carry):
        ds16 = pl.ds(k * 16, 16)
        qslot_v[ds16] = _slot_of(_hash_bits(qh_v[ds16]))
        qbest_v[ds16] = jnp.full((16,), 0x7FFFFFFF, jnp.int32)
        return carry
    lax.fori_loop(0, QPT // 16, qinit, 0)

    def probe_body(_n):
        pltpu.sync_copy(table_sh.at[qslot_v], qj_v)

        def g_step(i, carry):
            for u in range(8):
                ds16 = pl.ds(i * 128 + u * 16, 16)
                qgi_v[ds16] = jnp.maximum(qj_v[ds16], jnp.int32(0))
            return carry
        lax.fori_loop(0, QPT // 128, g_step, 0)
        pltpu.sync_copy(hk_hbm.at[qgi_v], qhg_v)

        def u_step(i, acc):
          for u in range(8):
            k = i * 8 + u
            ds16 = pl.ds(k * 16, 16)
            jv = qj_v[ds16]
            act = qbest_v[ds16] >= 0
            sl = qslot_v[ds16]
            bst = qbest_v[ds16] & 0x7FFFFFFF
            match = act & (jv != -1) & (qhg_v[ds16] == qh_v[ds16])
            bst = jnp.where(match, jnp.minimum(bst, jv), bst)
            act_new = act & (jv != -1)
            qbest_v[ds16] = jnp.where(act_new, bst, bst | jnp.int32(NEG0))
            nsl = sl + 1
            nsl = jnp.where(nsl >= M, nsl - M, nsl)
            qslot_v[ds16] = jnp.where(act_new, nsl, sl)
            acc = acc + jnp.where(act_new, jnp.int32(1), jnp.int32(0))
          return acc
        nact = lax.fori_loop(0, QPT // 128, u_step, jnp.zeros((16,), jnp.int32))
        return _scalar_total(nact)

    def probe_step(r, n):
        return lax.cond(n > 0, probe_body, lambda x: x, n)
    lax.fori_loop(0, MAX_ROUNDS, probe_step, jnp.int32(1))

    def f_step(k, carry):
        ds16 = pl.ds(k * 16, 16)
        qbest_v[ds16] = jnp.minimum(qbest_v[ds16] & 0x7FFFFFFF,
                                    jnp.int32(N_KEYS - 1))
        return carry
    lax.fori_loop(0, QPT // 16, f_step, 0)

    pltpu.sync_copy(d_hbm.at[qbest_v], qout_v)
    pltpu.sync_copy(qout_v, out_hbm.at[wid])


@jax.jit
def _sc_lookup(hk_pad, h_in, d_flat):
    mesh = plsc.VectorSubcoreMesh(core_axis_name="c", subcore_axis_name="s")
    f = pl.kernel(
        _lookup_body,
        out_type=jax.ShapeDtypeStruct((NC * NS, QPT), jnp.float32),
        mesh=mesh,
        scratch_types=[
            pltpu.VMEM((FILLW,), jnp.int32),
            pltpu.VMEM((KPT,), jnp.float32),
            pltpu.VMEM((KPT,), jnp.int32),
            pltpu.VMEM((KPT,), jnp.int32),
            pltpu.VMEM((KPT,), jnp.int32),
            pltpu.VMEM((KPT,), jnp.int32),
            pltpu.VMEM((KPT,), jnp.int32),
            pltpu.VMEM((16,), jnp.int32),
            pltpu.VMEM((NS * 16,), jnp.int32),
            pltpu.VMEM((QPT,), jnp.float32),
            pltpu.VMEM((QPT,), jnp.int32),
            pltpu.VMEM((QPT,), jnp.int32),
            pltpu.VMEM((QPT,), jnp.int32),
            pltpu.VMEM((QPT,), jnp.float32),
            pltpu.VMEM((QPT,), jnp.int32),
            pltpu.VMEM((QPT,), jnp.float32),
            pltpu.VMEM_SHARED((M + 128,), jnp.int32),
            pltpu.VMEM_SHARED((NS * 16,), jnp.int32),
        ],
    )
    return f(hk_pad, h_in, d_flat)


def kernel(inputs, keys_mat, distance_estimates, hash_vec):
    h_keys = jnp.sum(jnp.round(keys_mat, 5) * hash_vec, axis=-1)
    h_in = jnp.sum(jnp.round(inputs, 5) * hash_vec, axis=-1)
    hk_pad = jnp.pad(h_keys, (0, NPAD - N_KEYS))
    d_flat = distance_estimates[:, 0]
    out = _sc_lookup(hk_pad, h_in, d_flat)
    return out.reshape(BATCH, 1)


# speedup vs baseline: 2.2247x; 1.2852x over previous
import jax
import jax.numpy as jnp
from jax import lax
from jax.experimental import pallas as pl
from jax.experimental.pallas import tpu as pltpu
from jax.experimental.pallas import tpu_sc as plsc

N_KEYS = 100000
BATCH = 16384
NS = 16
NC = 2
KPT = 6272
NPAD = NS * KPT
QPT = BATCH // (NC * NS)
LOGM = 20
M = 1 << LOGM
HASH_MUL = -1640531527
NEG0 = -2147483648
FILLW = 16384
MAX_ROUNDS = 24


def _slot_of(hbits):
    return lax.shift_right_logical(hbits * HASH_MUL, 32 - LOGM)


def _hash_bits(h):
    return lax.convert_element_type(h * jnp.float32(1048576.0), jnp.int32)


def _scalar_total(vec):
    tot = jnp.int32(0)
    for l in range(16):
        tot = tot + vec[l]
    return tot


def _lookup_body(hk_hbm, hq_hbm, d_hbm, out_hbm,
                 neg1_v, hk_v, j_v, slot_v, won_v, wslot_v, rb_v,
                 cnt_v, cntall_v,
                 qh_v, qslot_v, qsl2_v, qj2_v, qgi2_v, qhg2_v, qbest_v, qout_v,
                 table_sh, cnt_sh):
    s = lax.axis_index("s")
    c = lax.axis_index("c")
    wid = s * NC + c

    def fill_body(i, carry):
        for u in range(8):
            neg1_v[pl.ds(i * 128 + u * 16, 16)] = jnp.full((16,), -1, jnp.int32)
        return carry
    lax.fori_loop(0, FILLW // 128, fill_body, 0)
    stripe = M // NS
    for r in range(stripe // FILLW):
        pltpu.sync_copy(neg1_v, table_sh.at[pl.ds(s * stripe + r * FILLW, FILLW)])

    pltpu.sync_copy(hk_hbm.at[pl.ds(s * KPT, KPT)], hk_v)

    def kinit(i, carry):
        for u in range(8):
            k = i * 8 + u
            ds16 = pl.ds(k * 16, 16)
            lane = lax.iota(jnp.int32, 16)
            j = s * KPT + k * 16 + lane
            sl = _slot_of(_hash_bits(hk_v[ds16]))
            valid = j < N_KEYS
            j_v[ds16] = j
            slot_v[ds16] = jnp.where(valid, sl, jnp.int32(M))
            won_v[ds16] = jnp.where(valid, jnp.int32(0), jnp.int32(1))
        return carry
    lax.fori_loop(0, KPT // 128, kinit, 0)

    plsc.subcore_barrier()

    def round_body(_tot):
        pltpu.sync_copy(table_sh.at[slot_v], rb_v)

        def a_step(i, carry):
            for u in range(8):
                ds16 = pl.ds(i * 128 + u * 16, 16)
                writer = (won_v[ds16] == 1) | (rb_v[ds16] == -1)
                wslot_v[ds16] = jnp.where(writer, slot_v[ds16], jnp.int32(M))
            return carry
        lax.fori_loop(0, KPT // 128, a_step, 0)

        plsc.subcore_barrier()
        pltpu.sync_copy(j_v, table_sh.at[wslot_v])
        plsc.subcore_barrier()
        pltpu.sync_copy(table_sh.at[wslot_v], rb_v)

        def b_step(i, acc):
            for u in range(8):
                ds16 = pl.ds(i * 128 + u * 16, 16)
                sl = slot_v[ds16]
                wn = (won_v[ds16] == 1) | ((wslot_v[ds16] != M) & (rb_v[ds16] == j_v[ds16]))
                nsl = sl + 1
                nsl = jnp.where(nsl >= M, nsl - M, nsl)
                slot_v[ds16] = jnp.where(wn, sl, nsl)
                won_v[ds16] = jnp.where(wn, jnp.int32(1), jnp.int32(0))
                acc = acc + jnp.where(wn, jnp.int32(0), jnp.int32(1))
            return acc
        pend = lax.fori_loop(0, KPT // 128, b_step, jnp.zeros((16,), jnp.int32))

        cnt_v[...] = pend
        pltpu.sync_copy(cnt_v, cnt_sh.at[pl.ds(s * 16, 16)])
        plsc.subcore_barrier()
        pltpu.sync_copy(cnt_sh, cntall_v)
        tot = jnp.zeros((16,), jnp.int32)
        for r in range(NS):
            tot = tot + cntall_v[pl.ds(r * 16, 16)]
        return _scalar_total(tot)

    def round_step(r, t):
        return lax.cond(t > 0, round_body, lambda x: x, t)
    lax.fori_loop(0, MAX_ROUNDS, round_step, jnp.int32(N_KEYS))

    pltpu.sync_copy(hq_hbm.at[pl.ds(wid * QPT, QPT)], qh_v)

    def qinit(k, carry):
        ds16 = pl.ds(k * 16, 16)
        qslot_v[ds16] = _slot_of(_hash_bits(qh_v[ds16]))
        qbest_v[ds16] = jnp.full((16,), 0x7FFFFFFF, jnp.int32)
        return carry
    lax.fori_loop(0, QPT // 16, qinit, 0)

    def probe_body(_n):
        def pi(i, carry):
            for u in range(8):
                ds16 = pl.ds(i * 128 + u * 16, 16)
                ds16b = pl.ds(QPT + i * 128 + u * 16, 16)
                sl = qslot_v[ds16]
                sl1 = sl + 1
                sl1 = jnp.where(sl1 >= M, sl1 - M, sl1)
                qsl2_v[ds16] = sl
                qsl2_v[ds16b] = sl1
            return carry
        lax.fori_loop(0, QPT // 128, pi, 0)
        pltpu.sync_copy(table_sh.at[qsl2_v], qj2_v)

        def pg(i, carry):
            for u in range(8):
                ds16 = pl.ds(i * 128 + u * 16, 16)
                qgi2_v[ds16] = jnp.maximum(qj2_v[ds16], jnp.int32(0))
            return carry
        lax.fori_loop(0, 2 * QPT // 128, pg, 0)
        pltpu.sync_copy(hk_hbm.at[qgi2_v], qhg2_v)

        def pu(i, acc):
            for u in range(8):
                k = i * 8 + u
                ds16 = pl.ds(k * 16, 16)
                ds16b = pl.ds(QPT + k * 16, 16)
                j0 = qj2_v[ds16]
                j1 = qj2_v[ds16b]
                h0 = qhg2_v[ds16]
                h1 = qhg2_v[ds16b]
                hq = qh_v[ds16]
                act = qbest_v[ds16] >= 0
                sl = qslot_v[ds16]
                bst = qbest_v[ds16] & 0x7FFFFFFF
                match0 = act & (j0 != -1) & (h0 == hq)
                bst = jnp.where(match0, jnp.minimum(bst, j0), bst)
                act0 = act & (j0 != -1)
                match1 = act0 & (j1 != -1) & (h1 == hq)
                bst = jnp.where(match1, jnp.minimum(bst, j1), bst)
                act_new = act0 & (j1 != -1)
                qbest_v[ds16] = jnp.where(act_new, bst, bst | jnp.int32(NEG0))
                nsl = sl + 2
                nsl = jnp.where(nsl >= M, nsl - M, nsl)
                qslot_v[ds16] = jnp.where(act_new, nsl, sl)
                acc = acc + jnp.where(act_new, jnp.int32(1), jnp.int32(0))
            return acc
        nact = lax.fori_loop(0, QPT // 128, pu, jnp.zeros((16,), jnp.int32))
        return _scalar_total(nact)

    def probe_step(r, n):
        return lax.cond(n > 0, probe_body, lambda x: x, n)
    lax.fori_loop(0, MAX_ROUNDS, probe_step, jnp.int32(1))

    def f_step(k, carry):
        ds16 = pl.ds(k * 16, 16)
        qbest_v[ds16] = jnp.minimum(qbest_v[ds16] & 0x7FFFFFFF,
                                    jnp.int32(N_KEYS - 1))
        return carry
    lax.fori_loop(0, QPT // 16, f_step, 0)

    pltpu.sync_copy(d_hbm.at[qbest_v], qout_v)
    pltpu.sync_copy(qout_v, out_hbm.at[wid])


@jax.jit
def _sc_lookup(hk_pad, h_in, d_flat):
    mesh = plsc.VectorSubcoreMesh(core_axis_name="c", subcore_axis_name="s")
    f = pl.kernel(
        _lookup_body,
        out_type=jax.ShapeDtypeStruct((NC * NS, QPT), jnp.float32),
        mesh=mesh,
        scratch_types=[
            pltpu.VMEM((FILLW,), jnp.int32),
            pltpu.VMEM((KPT,), jnp.float32),
            pltpu.VMEM((KPT,), jnp.int32),
            pltpu.VMEM((KPT,), jnp.int32),
            pltpu.VMEM((KPT,), jnp.int32),
            pltpu.VMEM((KPT,), jnp.int32),
            pltpu.VMEM((KPT,), jnp.int32),
            pltpu.VMEM((16,), jnp.int32),
            pltpu.VMEM((NS * 16,), jnp.int32),
            pltpu.VMEM((QPT,), jnp.float32),
            pltpu.VMEM((QPT,), jnp.int32),
            pltpu.VMEM((2 * QPT,), jnp.int32),
            pltpu.VMEM((2 * QPT,), jnp.int32),
            pltpu.VMEM((2 * QPT,), jnp.int32),
            pltpu.VMEM((2 * QPT,), jnp.float32),
            pltpu.VMEM((QPT,), jnp.int32),
            pltpu.VMEM((QPT,), jnp.float32),
            pltpu.VMEM_SHARED((M + 128,), jnp.int32),
            pltpu.VMEM_SHARED((NS * 16,), jnp.int32),
        ],
    )
    return f(hk_pad, h_in, d_flat)


def kernel(inputs, keys_mat, distance_estimates, hash_vec):
    h_keys = jnp.sum(jnp.round(keys_mat, 5) * hash_vec, axis=-1)
    h_in = jnp.sum(jnp.round(inputs, 5) * hash_vec, axis=-1)
    hk_pad = jnp.pad(h_keys, (0, NPAD - N_KEYS))
    d_flat = distance_estimates[:, 0]
    out = _sc_lookup(hk_pad, h_in, d_flat)
    return out.reshape(BATCH, 1)
